# Initial kernel scaffold; baseline (speedup 1.0000x reference)
import jax
import jax.numpy as jnp
from jax.experimental import pallas as pl


def kernel(input, segLabels):
    raise NotImplementedError("write your pallas kernel here")


# speedup vs baseline: 5.3491x; 5.3491x over previous
import functools

import jax
import jax.numpy as jnp
from jax import lax
from jax.experimental import pallas as pl
from jax.experimental.pallas import tpu as pltpu
from jax.experimental.pallas import tpu_sc as plsc

N = 320000
D = 128
NUM_SEG = 10000
SEG_PAD = 10240
GROUP = 128
NGROUPS = N // GROUP
LANES = 16

_info = plsc.get_sparse_core_info()
NC = _info.num_cores
NS = _info.num_subcores
NW = NC * NS
SEG_PER_TILE = SEG_PAD // NS


def _phase1_body(x_hbm, seg_hbm, sums_hbm, cnts_hbm,
                 dbuf, idx_v, ones_v, zrow, zcnt, acc_sp, cnt_sp):
    c = lax.axis_index("c")
    s = lax.axis_index("s")
    wid = s * NC + c

    zero16 = jnp.zeros((LANES,), jnp.float32)
    one16 = jnp.ones((LANES,), jnp.float32)

    def init_zrow(i, _):
        for j in range(D // LANES):
            zrow[i, pl.ds(j * LANES, LANES)] = zero16
        return 0
    lax.fori_loop(0, zrow.shape[0], init_zrow, 0)

    def init_zcnt(i, _):
        zcnt[pl.ds(i * LANES, LANES)] = zero16
        return 0
    lax.fori_loop(0, SEG_PER_TILE // LANES, init_zcnt, 0)

    def init_ones(i, _):
        ones_v[pl.ds(i * LANES, LANES)] = one16
        return 0
    lax.fori_loop(0, GROUP // LANES, init_ones, 0)

    zr = zrow.shape[0]
    for r in range(SEG_PER_TILE // zr):
        pltpu.sync_copy(zrow, acc_sp.at[pl.ds(s * SEG_PER_TILE + r * zr, zr)])
    pltpu.sync_copy(zcnt, cnt_sp.at[pl.ds(s * SEG_PER_TILE, SEG_PER_TILE)])
    plsc.subcore_barrier()

    gs = (wid * NGROUPS) // NW
    ge = ((wid + 1) * NGROUPS) // NW

    def body(g, _):
        off = g * GROUP
        pltpu.sync_copy(seg_hbm.at[pl.ds(off, GROUP)], idx_v)
        pltpu.sync_copy(x_hbm.at[pl.ds(off, GROUP)], dbuf)
        pltpu.sync_copy(dbuf, acc_sp.at[idx_v], add=True)
        pltpu.sync_copy(ones_v, cnt_sp.at[idx_v], add=True)
        return 0
    lax.fori_loop(gs, ge, body, 0)

    plsc.subcore_barrier()

    lo = s * SEG_PER_TILE
    pltpu.sync_copy(acc_sp.at[pl.ds(lo, SEG_PER_TILE)],
                    sums_hbm.at[c].at[pl.ds(lo, SEG_PER_TILE)])
    pltpu.sync_copy(cnt_sp.at[pl.ds(lo, SEG_PER_TILE)],
                    cnts_hbm.at[c].at[pl.ds(lo, SEG_PER_TILE)])


_phase1 = functools.partial(
    pl.kernel,
    mesh=plsc.VectorSubcoreMesh(core_axis_name="c", subcore_axis_name="s"),
    out_type=[
        jax.ShapeDtypeStruct((NC, SEG_PAD, D), jnp.float32),
        jax.ShapeDtypeStruct((NC, SEG_PAD), jnp.float32),
    ],
    scratch_types=[
        pltpu.VMEM((GROUP, D), jnp.float32),
        pltpu.VMEM((GROUP,), jnp.int32),
        pltpu.VMEM((GROUP,), jnp.float32),
        pltpu.VMEM((GROUP, D), jnp.float32),
        pltpu.VMEM((SEG_PER_TILE,), jnp.float32),
        pltpu.VMEM_SHARED((SEG_PAD, D), jnp.float32),
        pltpu.VMEM_SHARED((SEG_PAD,), jnp.float32),
    ],
)(_phase1_body)


def _phase2_body(s0, s1, c0, c1, o):
    cnt = c0[...] + c1[...]
    o[...] = (s0[...] + s1[...]) / jnp.maximum(cnt, 1.0)


_BS = 400

_phase2 = pl.pallas_call(
    _phase2_body,
    grid=(NUM_SEG // _BS,),
    in_specs=[
        pl.BlockSpec((_BS, D), lambda i: (i, 0)),
        pl.BlockSpec((_BS, D), lambda i: (i, 0)),
        pl.BlockSpec((_BS, 1), lambda i: (i, 0)),
        pl.BlockSpec((_BS, 1), lambda i: (i, 0)),
    ],
    out_specs=pl.BlockSpec((_BS, D), lambda i: (i, 0)),
    out_shape=jax.ShapeDtypeStruct((NUM_SEG, D), jnp.float32),
)


@jax.jit
def kernel(input, segLabels):
    seg = segLabels.astype(jnp.int32)
    sums, cnts = _phase1(input, seg)
    cn = cnts[:, :, None]
    return _phase2(sums[0], sums[1], cn[0], cn[1])


# speedup vs baseline: 8.2886x; 1.5495x over previous
import functools

import jax
import jax.numpy as jnp
from jax import lax
from jax.experimental import pallas as pl
from jax.experimental.pallas import tpu as pltpu
from jax.experimental.pallas import tpu_sc as plsc

N = 320000
D = 128
NUM_SEG = 10000
SEG_PAD = 10240
GROUP = 128
NGROUPS = N // GROUP
LANES = 16

_info = plsc.get_sparse_core_info()
NC = _info.num_cores
NS = _info.num_subcores
NW = NC * NS
SEG_PER_TILE = SEG_PAD // NS
TRIPS = -(-NGROUPS // NW)
TRIPS += TRIPS % 2


def _phase1_body(x_hbm, seg_hbm, sums_hbm, cnts_hbm,
                 dbuf0, dbuf1, idx0, idx1, ones_v, zcnt, acc_sp, cnt_sp,
                 sem_d0, sem_i0, sem_d1, sem_i1, sem_s, sem_c):
    c = lax.axis_index("c")
    s = lax.axis_index("s")
    wid = s * NC + c

    zero16 = jnp.zeros((LANES,), jnp.float32)
    one16 = jnp.ones((LANES,), jnp.float32)

    def init_zrow(i, _):
        for j in range(D // LANES):
            dbuf0[i, pl.ds(j * LANES, LANES)] = zero16
        return 0
    lax.fori_loop(0, GROUP, init_zrow, 0)

    def init_zcnt(i, _):
        zcnt[pl.ds(i * LANES, LANES)] = zero16
        return 0
    lax.fori_loop(0, SEG_PER_TILE // LANES, init_zcnt, 0)

    def init_ones(i, _):
        ones_v[pl.ds(i * LANES, LANES)] = one16
        return 0
    lax.fori_loop(0, GROUP // LANES, init_ones, 0)

    zr = GROUP
    for r in range(SEG_PER_TILE // zr):
        pltpu.sync_copy(dbuf0, acc_sp.at[pl.ds(s * SEG_PER_TILE + r * zr, zr)])
    pltpu.sync_copy(zcnt, cnt_sp.at[pl.ds(s * SEG_PER_TILE, SEG_PER_TILE)])
    plsc.subcore_barrier()

    gs = (wid * NGROUPS) // NW
    ge = ((wid + 1) * NGROUPS) // NW
    ng = ge - gs

    slots = ((dbuf0, idx0, sem_d0, sem_i0), (dbuf1, idx1, sem_d1, sem_i1))

    def start(i, slot):
        dbuf_b, idx_b, sem_d, sem_i = slot
        gi = jnp.where(i < ng, gs + i, gs)
        off = gi * GROUP
        pltpu.async_copy(x_hbm.at[pl.ds(off, GROUP)], dbuf_b, sem_d)
        pltpu.async_copy(seg_hbm.at[pl.ds(off, GROUP)], idx_b, sem_i)

    def finish(i, slot):
        dbuf_b, idx_b, sem_d, sem_i = slot
        pltpu.make_async_copy(seg_hbm.at[pl.ds(0, GROUP)], idx_b, sem_i).wait()

        @pl.when(i >= ng)
        def _():
            pad = jnp.full((LANES,), NUM_SEG, jnp.int32)
            for j in range(GROUP // LANES):
                idx_b[pl.ds(j * LANES, LANES)] = pad

        pltpu.make_async_copy(x_hbm.at[pl.ds(0, GROUP)], dbuf_b, sem_d).wait()
        hd = pltpu.async_copy(dbuf_b, acc_sp.at[idx_b], sem_s, add=True)
        hc = pltpu.async_copy(ones_v, cnt_sp.at[idx_b], sem_c, add=True)
        hd.wait()
        hc.wait()

    start(0, slots[0])

    def pair(it, _):
        base = 2 * it
        start(base + 1, slots[1])
        finish(base, slots[0])

        @pl.when(base + 2 < TRIPS)
        def _():
            start(base + 2, slots[0])

        finish(base + 1, slots[1])
        return 0
    lax.fori_loop(0, TRIPS // 2, pair, 0)

    plsc.subcore_barrier()

    lo = s * SEG_PER_TILE
    pltpu.sync_copy(acc_sp.at[pl.ds(lo, SEG_PER_TILE)],
                    sums_hbm.at[c].at[pl.ds(lo, SEG_PER_TILE)])
    pltpu.sync_copy(cnt_sp.at[pl.ds(lo, SEG_PER_TILE)],
                    cnts_hbm.at[c].at[pl.ds(lo, SEG_PER_TILE)])


_phase1 = functools.partial(
    pl.kernel,
    mesh=plsc.VectorSubcoreMesh(core_axis_name="c", subcore_axis_name="s"),
    out_type=[
        jax.ShapeDtypeStruct((NC, SEG_PAD, D), jnp.float32),
        jax.ShapeDtypeStruct((NC, SEG_PAD), jnp.float32),
    ],
    scratch_types=[
        pltpu.VMEM((GROUP, D), jnp.float32),
        pltpu.VMEM((GROUP, D), jnp.float32),
        pltpu.VMEM((GROUP,), jnp.int32),
        pltpu.VMEM((GROUP,), jnp.int32),
        pltpu.VMEM((GROUP,), jnp.float32),
        pltpu.VMEM((SEG_PER_TILE,), jnp.float32),
        pltpu.VMEM_SHARED((SEG_PAD, D), jnp.float32),
        pltpu.VMEM_SHARED((SEG_PAD,), jnp.float32),
        pltpu.SemaphoreType.DMA,
        pltpu.SemaphoreType.DMA,
        pltpu.SemaphoreType.DMA,
        pltpu.SemaphoreType.DMA,
        pltpu.SemaphoreType.DMA,
        pltpu.SemaphoreType.DMA,
    ],
)(_phase1_body)


def _phase2_body(s0, s1, c0, c1, o):
    cnt = c0[...] + c1[...]
    o[...] = (s0[...] + s1[...]) / jnp.maximum(cnt, 1.0)


_BS = 400

_phase2 = pl.pallas_call(
    _phase2_body,
    grid=(NUM_SEG // _BS,),
    in_specs=[
        pl.BlockSpec((_BS, D), lambda i: (i, 0)),
        pl.BlockSpec((_BS, D), lambda i: (i, 0)),
        pl.BlockSpec((_BS, 1), lambda i: (i, 0)),
        pl.BlockSpec((_BS, 1), lambda i: (i, 0)),
    ],
    out_specs=pl.BlockSpec((_BS, D), lambda i: (i, 0)),
    out_shape=jax.ShapeDtypeStruct((NUM_SEG, D), jnp.float32),
)


@jax.jit
def kernel(input, segLabels):
    seg = segLabels.astype(jnp.int32)
    sums, cnts = _phase1(input, seg)
    cn = cnts[:, :, None]
    return _phase2(sums[0], sums[1], cn[0], cn[1])


# speedup vs baseline: 8.3372x; 1.0059x over previous
import functools

import jax
import jax.numpy as jnp
from jax import lax
from jax.experimental import pallas as pl
from jax.experimental.pallas import tpu as pltpu
from jax.experimental.pallas import tpu_sc as plsc

N = 320000
D = 128
NUM_SEG = 10000
SEG_PAD = 10240
GROUP = 128
NGROUPS = N // GROUP
LANES = 16

_info = plsc.get_sparse_core_info()
NC = _info.num_cores
NS = _info.num_subcores
NW = NC * NS
SEG_PER_TILE = SEG_PAD // NS
TRIPS = -(-NGROUPS // NW)
TRIPS += TRIPS % 2


def _phase1_body(x_hbm, seg_hbm, sums_hbm, cnts_hbm,
                 dbuf0, dbuf1, idx0, idx1, ones_v, zcnt, acc_sp, cnt_sp,
                 sem_d0, sem_i0, sem_d1, sem_i1, sem_s, sem_c):
    c = lax.axis_index("c")
    s = lax.axis_index("s")
    wid = s * NC + c

    zero16 = jnp.zeros((LANES,), jnp.float32)
    one16 = jnp.ones((LANES,), jnp.float32)

    def init_zrow(i, _):
        for j in range(D // LANES):
            dbuf0[i, pl.ds(j * LANES, LANES)] = zero16
        return 0
    lax.fori_loop(0, GROUP, init_zrow, 0)

    def init_zcnt(i, _):
        zcnt[pl.ds(i * LANES, LANES)] = zero16
        return 0
    lax.fori_loop(0, SEG_PER_TILE // LANES, init_zcnt, 0)

    def init_ones(i, _):
        ones_v[pl.ds(i * LANES, LANES)] = one16
        return 0
    lax.fori_loop(0, GROUP // LANES, init_ones, 0)

    zr = GROUP
    for r in range(SEG_PER_TILE // zr):
        pltpu.sync_copy(dbuf0, acc_sp.at[pl.ds(s * SEG_PER_TILE + r * zr, zr)])
    pltpu.sync_copy(zcnt, cnt_sp.at[pl.ds(s * SEG_PER_TILE, SEG_PER_TILE)])
    plsc.subcore_barrier()

    gs = (wid * NGROUPS) // NW
    ge = ((wid + 1) * NGROUPS) // NW
    ng = ge - gs

    slots = ((dbuf0, idx0, sem_d0, sem_i0), (dbuf1, idx1, sem_d1, sem_i1))

    def start(i, slot):
        dbuf_b, idx_b, sem_d, sem_i = slot
        gi = jnp.where(i < ng, gs + i, gs)
        off = gi * GROUP
        pltpu.async_copy(x_hbm.at[pl.ds(off, GROUP)], dbuf_b, sem_d)
        pltpu.async_copy(seg_hbm.at[pl.ds(off, GROUP)], idx_b, sem_i)

    def finish(i, slot):
        dbuf_b, idx_b, sem_d, sem_i = slot
        pltpu.make_async_copy(seg_hbm.at[pl.ds(0, GROUP)], idx_b, sem_i).wait()

        @pl.when(i >= ng)
        def _():
            pad = jnp.full((LANES,), NUM_SEG, jnp.int32)
            for j in range(GROUP // LANES):
                idx_b[pl.ds(j * LANES, LANES)] = pad

        pltpu.make_async_copy(x_hbm.at[pl.ds(0, GROUP)], dbuf_b, sem_d).wait()
        pltpu.sync_copy(dbuf_b, acc_sp.at[idx_b], add=True)
        pltpu.sync_copy(ones_v, cnt_sp.at[idx_b], add=True)

    start(0, slots[0])

    def pair(it, _):
        base = 2 * it
        start(base + 1, slots[1])
        finish(base, slots[0])

        @pl.when(base + 2 < TRIPS)
        def _():
            start(base + 2, slots[0])

        finish(base + 1, slots[1])
        return 0
    lax.fori_loop(0, TRIPS // 2, pair, 0)

    plsc.subcore_barrier()

    lo = s * SEG_PER_TILE
    pltpu.sync_copy(acc_sp.at[pl.ds(lo, SEG_PER_TILE)],
                    sums_hbm.at[c].at[pl.ds(lo, SEG_PER_TILE)])
    pltpu.sync_copy(cnt_sp.at[pl.ds(lo, SEG_PER_TILE)],
                    cnts_hbm.at[c].at[pl.ds(lo, SEG_PER_TILE)])


_phase1 = functools.partial(
    pl.kernel,
    mesh=plsc.VectorSubcoreMesh(core_axis_name="c", subcore_axis_name="s"),
    out_type=[
        jax.ShapeDtypeStruct((NC, SEG_PAD, D), jnp.float32),
        jax.ShapeDtypeStruct((NC, SEG_PAD), jnp.float32),
    ],
    scratch_types=[
        pltpu.VMEM((GROUP, D), jnp.float32),
        pltpu.VMEM((GROUP, D), jnp.float32),
        pltpu.VMEM((GROUP,), jnp.int32),
        pltpu.VMEM((GROUP,), jnp.int32),
        pltpu.VMEM((GROUP,), jnp.float32),
        pltpu.VMEM((SEG_PER_TILE,), jnp.float32),
        pltpu.VMEM_SHARED((SEG_PAD, D), jnp.float32),
        pltpu.VMEM_SHARED((SEG_PAD,), jnp.float32),
        pltpu.SemaphoreType.DMA,
        pltpu.SemaphoreType.DMA,
        pltpu.SemaphoreType.DMA,
        pltpu.SemaphoreType.DMA,
        pltpu.SemaphoreType.DMA,
        pltpu.SemaphoreType.DMA,
    ],
)(_phase1_body)


def _phase2_body(s0, s1, c0, c1, o):
    cnt = c0[...] + c1[...]
    o[...] = (s0[...] + s1[...]) / jnp.maximum(cnt, 1.0)


_BS = 400

_phase2 = pl.pallas_call(
    _phase2_body,
    grid=(NUM_SEG // _BS,),
    in_specs=[
        pl.BlockSpec((_BS, D), lambda i: (i, 0)),
        pl.BlockSpec((_BS, D), lambda i: (i, 0)),
        pl.BlockSpec((_BS, 1), lambda i: (i, 0)),
        pl.BlockSpec((_BS, 1), lambda i: (i, 0)),
    ],
    out_specs=pl.BlockSpec((_BS, D), lambda i: (i, 0)),
    out_shape=jax.ShapeDtypeStruct((NUM_SEG, D), jnp.float32),
)


@jax.jit
def kernel(input, segLabels):
    seg = segLabels.astype(jnp.int32)
    sums, cnts = _phase1(input, seg)
    cn = cnts[:, :, None]
    return _phase2(sums[0], sums[1], cn[0], cn[1])


# speedup vs baseline: 8.7157x; 1.0454x over previous
import functools

import jax
import jax.numpy as jnp
from jax import lax
from jax.experimental import pallas as pl
from jax.experimental.pallas import tpu as pltpu
from jax.experimental.pallas import tpu_sc as plsc

N = 320000
D = 128
NUM_SEG = 10000
SEG_PAD = 10240
GROUP = 128
NGROUPS = N // GROUP
LANES = 16

_info = plsc.get_sparse_core_info()
NC = _info.num_cores
NS = _info.num_subcores
NW = NC * NS
SEG_PER_TILE = SEG_PAD // NS
TRIPS = -(-NGROUPS // NW)
TRIPS += TRIPS % 2


def _phase1_body(x_hbm, seg_hbm, zsum_hbm, sums_hbm, cnts_hbm,
                 dbuf0, dbuf1, idx0, idx1, ones_v, zcnt, acc_sp, cnt_sp,
                 sem_d0, sem_i0, sem_d1, sem_i1):
    c = lax.axis_index("c")
    s = lax.axis_index("s")
    wid = s * NC + c

    zero16 = jnp.zeros((LANES,), jnp.float32)
    one16 = jnp.ones((LANES,), jnp.float32)

    def init_zcnt(i, _):
        zcnt[pl.ds(i * LANES, LANES)] = zero16
        return 0
    lax.fori_loop(0, SEG_PER_TILE // LANES, init_zcnt, 0)

    def init_ones(i, _):
        ones_v[pl.ds(i * LANES, LANES)] = one16
        return 0
    lax.fori_loop(0, GROUP // LANES, init_ones, 0)

    gs = (wid * NGROUPS) // NW
    ge = ((wid + 1) * NGROUPS) // NW
    ng = ge - gs

    slots = ((dbuf0, idx0, sem_d0, sem_i0), (dbuf1, idx1, sem_d1, sem_i1))

    def start(i, slot):
        dbuf_b, idx_b, sem_d, sem_i = slot
        gi = jnp.where(i < ng, gs + i, gs)
        off = gi * GROUP
        pltpu.async_copy(x_hbm.at[pl.ds(off, GROUP)], dbuf_b, sem_d)
        pltpu.async_copy(seg_hbm.at[pl.ds(off, GROUP)], idx_b, sem_i)

    def finish(i, slot):
        dbuf_b, idx_b, sem_d, sem_i = slot
        pltpu.make_async_copy(seg_hbm.at[pl.ds(0, GROUP)], idx_b, sem_i).wait()

        @pl.when(i >= ng)
        def _():
            pad = jnp.full((LANES,), NUM_SEG, jnp.int32)
            for j in range(GROUP // LANES):
                idx_b[pl.ds(j * LANES, LANES)] = pad

        pltpu.make_async_copy(x_hbm.at[pl.ds(0, GROUP)], dbuf_b, sem_d).wait()
        pltpu.sync_copy(dbuf_b, acc_sp.at[idx_b], add=True)
        pltpu.sync_copy(ones_v, cnt_sp.at[idx_b], add=True)

    start(0, slots[0])
    start(1, slots[1])

    lo = s * SEG_PER_TILE
    pltpu.sync_copy(zsum_hbm.at[pl.ds(lo, SEG_PER_TILE)],
                    acc_sp.at[pl.ds(lo, SEG_PER_TILE)])
    pltpu.sync_copy(zcnt, cnt_sp.at[pl.ds(lo, SEG_PER_TILE)])
    plsc.subcore_barrier()

    def pair(it, _):
        base = 2 * it
        finish(base, slots[0])

        @pl.when(base + 2 < TRIPS)
        def _():
            start(base + 2, slots[0])

        finish(base + 1, slots[1])

        @pl.when(base + 3 < TRIPS)
        def _():
            start(base + 3, slots[1])
        return 0
    lax.fori_loop(0, TRIPS // 2, pair, 0)

    plsc.subcore_barrier()

    pltpu.sync_copy(acc_sp.at[pl.ds(lo, SEG_PER_TILE)],
                    sums_hbm.at[c].at[pl.ds(lo, SEG_PER_TILE)])
    pltpu.sync_copy(cnt_sp.at[pl.ds(lo, SEG_PER_TILE)],
                    cnts_hbm.at[c].at[pl.ds(lo, SEG_PER_TILE)])


_phase1 = functools.partial(
    pl.kernel,
    mesh=plsc.VectorSubcoreMesh(core_axis_name="c", subcore_axis_name="s"),
    out_type=[
        jax.ShapeDtypeStruct((NC, SEG_PAD, D), jnp.float32),
        jax.ShapeDtypeStruct((NC, SEG_PAD), jnp.float32),
    ],
    scratch_types=[
        pltpu.VMEM((GROUP, D), jnp.float32),
        pltpu.VMEM((GROUP, D), jnp.float32),
        pltpu.VMEM((GROUP,), jnp.int32),
        pltpu.VMEM((GROUP,), jnp.int32),
        pltpu.VMEM((GROUP,), jnp.float32),
        pltpu.VMEM((SEG_PER_TILE,), jnp.float32),
        pltpu.VMEM_SHARED((SEG_PAD, D), jnp.float32),
        pltpu.VMEM_SHARED((SEG_PAD,), jnp.float32),
        pltpu.SemaphoreType.DMA,
        pltpu.SemaphoreType.DMA,
        pltpu.SemaphoreType.DMA,
        pltpu.SemaphoreType.DMA,
    ],
)(_phase1_body)


def _phase2_body(s0, s1, c0, c1, o):
    cnt = c0[...] + c1[...]
    o[...] = (s0[...] + s1[...]) / jnp.maximum(cnt, 1.0)


_BS = 2000

_phase2 = pl.pallas_call(
    _phase2_body,
    grid=(NUM_SEG // _BS,),
    in_specs=[
        pl.BlockSpec((_BS, D), lambda i: (i, 0)),
        pl.BlockSpec((_BS, D), lambda i: (i, 0)),
        pl.BlockSpec((_BS, 1), lambda i: (i, 0)),
        pl.BlockSpec((_BS, 1), lambda i: (i, 0)),
    ],
    out_specs=pl.BlockSpec((_BS, D), lambda i: (i, 0)),
    out_shape=jax.ShapeDtypeStruct((NUM_SEG, D), jnp.float32),
)


@jax.jit
def kernel(input, segLabels):
    seg = segLabels.astype(jnp.int32)
    zsum = jnp.zeros((SEG_PAD, D), jnp.float32)
    sums, cnts = _phase1(input, seg, zsum)
    cn = cnts[:, :, None]
    return _phase2(sums[0], sums[1], cn[0], cn[1])


# speedup vs baseline: 9.1406x; 1.0487x over previous
import functools

import jax
import jax.numpy as jnp
from jax import lax
from jax.experimental import pallas as pl
from jax.experimental.pallas import tpu as pltpu
from jax.experimental.pallas import tpu_sc as plsc

N = 320000
D = 128
NUM_SEG = 10000
SEG_PAD = 10240
GROUP = 128
NGROUPS = N // GROUP
LANES = 16

_info = plsc.get_sparse_core_info()
NC = _info.num_cores
NS = _info.num_subcores
NW = NC * NS
SEG_PER_TILE = SEG_PAD // NS
TRIPS = -(-NGROUPS // NW)
TRIPS += TRIPS % 2


def _phase1_body(x_hbm, seg_hbm, zsum_hbm, sums_hbm, cnts_hbm,
                 dbuf0, dbuf1, idx0, idx1, ones_v, zcnt, acc_sp, cnt_sp,
                 sem_d0, sem_i0, sem_d1, sem_i1):
    c = lax.axis_index("c")
    s = lax.axis_index("s")
    wid = s * NC + c

    zero16 = jnp.zeros((LANES,), jnp.float32)
    one16 = jnp.ones((LANES,), jnp.float32)

    def init_zcnt(i, _):
        zcnt[pl.ds(i * LANES, LANES)] = zero16
        return 0
    lax.fori_loop(0, SEG_PER_TILE // LANES, init_zcnt, 0)

    def init_ones(i, _):
        ones_v[pl.ds(i * LANES, LANES)] = one16
        return 0
    lax.fori_loop(0, GROUP // LANES, init_ones, 0)

    gs = (wid * NGROUPS) // NW
    ge = ((wid + 1) * NGROUPS) // NW
    ng = ge - gs

    slots = ((dbuf0, idx0, sem_d0, sem_i0), (dbuf1, idx1, sem_d1, sem_i1))

    def start(i, slot):
        dbuf_b, idx_b, sem_d, sem_i = slot
        gi = jnp.where(i < ng, gs + i, gs)
        off = gi * GROUP
        pltpu.async_copy(x_hbm.at[pl.ds(off, GROUP)], dbuf_b, sem_d)
        pltpu.async_copy(seg_hbm.at[pl.ds(off, GROUP)], idx_b, sem_i)

    def finish(i, slot):
        dbuf_b, idx_b, sem_d, sem_i = slot
        pltpu.make_async_copy(seg_hbm.at[pl.ds(0, GROUP)], idx_b, sem_i).wait()

        @pl.when(i >= ng)
        def _():
            pad = jnp.full((LANES,), NUM_SEG, jnp.int32)
            for j in range(GROUP // LANES):
                idx_b[pl.ds(j * LANES, LANES)] = pad

        pltpu.make_async_copy(x_hbm.at[pl.ds(0, GROUP)], dbuf_b, sem_d).wait()
        pltpu.sync_copy(dbuf_b, acc_sp.at[idx_b], add=True)
        pltpu.sync_copy(ones_v, cnt_sp.at[idx_b], add=True)

    start(0, slots[0])
    start(1, slots[1])

    lo = s * SEG_PER_TILE
    pltpu.sync_copy(zsum_hbm.at[pl.ds(lo, SEG_PER_TILE)],
                    acc_sp.at[pl.ds(lo, SEG_PER_TILE)])
    pltpu.sync_copy(zcnt, cnt_sp.at[pl.ds(lo, SEG_PER_TILE)])
    plsc.subcore_barrier()

    def pair(it, _):
        base = 2 * it
        finish(base, slots[0])

        @pl.when(base + 2 < TRIPS)
        def _():
            start(base + 2, slots[0])

        finish(base + 1, slots[1])

        @pl.when(base + 3 < TRIPS)
        def _():
            start(base + 3, slots[1])
        return 0
    lax.fori_loop(0, TRIPS // 2, pair, 0)

    plsc.subcore_barrier()

    pltpu.sync_copy(acc_sp.at[pl.ds(lo, SEG_PER_TILE)],
                    sums_hbm.at[c].at[pl.ds(lo, SEG_PER_TILE)])
    pltpu.sync_copy(cnt_sp.at[pl.ds(lo, SEG_PER_TILE)],
                    cnts_hbm.at[c].at[pl.ds(lo, SEG_PER_TILE)])


_phase1 = functools.partial(
    pl.kernel,
    mesh=plsc.VectorSubcoreMesh(core_axis_name="c", subcore_axis_name="s"),
    out_type=[
        jax.ShapeDtypeStruct((NC, SEG_PAD, D), jnp.float32),
        jax.ShapeDtypeStruct((NC, SEG_PAD), jnp.float32),
    ],
    scratch_types=[
        pltpu.VMEM((GROUP, D), jnp.float32),
        pltpu.VMEM((GROUP, D), jnp.float32),
        pltpu.VMEM((GROUP,), jnp.int32),
        pltpu.VMEM((GROUP,), jnp.int32),
        pltpu.VMEM((GROUP,), jnp.float32),
        pltpu.VMEM((SEG_PER_TILE,), jnp.float32),
        pltpu.VMEM_SHARED((SEG_PAD, D), jnp.float32),
        pltpu.VMEM_SHARED((SEG_PAD,), jnp.float32),
        pltpu.SemaphoreType.DMA,
        pltpu.SemaphoreType.DMA,
        pltpu.SemaphoreType.DMA,
        pltpu.SemaphoreType.DMA,
    ],
)(_phase1_body)


def _phase2_body(s0, s1, cc, o):
    c = cc[...]
    cnt = jnp.transpose(c[0:1, :] + c[1:2, :], (1, 0))
    o[...] = (s0[...] + s1[...]) / jnp.maximum(cnt, 1.0)


_BS = 2048

_phase2 = pl.pallas_call(
    _phase2_body,
    grid=(SEG_PAD // _BS,),
    in_specs=[
        pl.BlockSpec((_BS, D), lambda i: (i, 0)),
        pl.BlockSpec((_BS, D), lambda i: (i, 0)),
        pl.BlockSpec((NC, _BS), lambda i: (0, i)),
    ],
    out_specs=pl.BlockSpec((_BS, D), lambda i: (i, 0)),
    out_shape=jax.ShapeDtypeStruct((NUM_SEG, D), jnp.float32),
)


@jax.jit
def kernel(input, segLabels):
    seg = segLabels.astype(jnp.int32)
    zsum = jnp.zeros((SEG_PAD, D), jnp.float32)
    sums, cnts = _phase1(input, seg, zsum)
    return _phase2(sums[0], sums[1], cnts)
